# baseline (speedup 1.0000x reference)
import jax
import jax.numpy as jnp
from jax.experimental import pallas as pl
from jax.experimental.pallas import tpu as pltpu

_SHIFT = 8192

_DEPTH = 8
_AHEAD = 4


def _make_copy(s, b, c, out_len, dtype):
    tasks = [(si, bi) for si in range(s) for bi in range(b)]
    ntask = len(tasks)

    def body(in_ref, out_ref, *rest):
        bufs, (rsem, wsem) = rest[:_DEPTH], rest[_DEPTH:]

        def read_copy(t):
            si, bi = tasks[t]
            slot = t % _DEPTH
            return pltpu.make_async_copy(
                in_ref.at[si, bi, :, pl.ds(0, out_len)],
                bufs[slot],
                rsem.at[slot],
            )

        def write_copy(t):
            si, bi = tasks[t]
            slot = t % _DEPTH
            return pltpu.make_async_copy(
                bufs[slot],
                out_ref.at[si, bi],
                wsem.at[slot],
            )

        for t in range(min(_AHEAD, ntask)):
            read_copy(t).start()
        for t in range(ntask):
            nt = t + _AHEAD
            if nt < ntask:
                if nt >= _DEPTH:
                    write_copy(nt - _DEPTH).wait()
                read_copy(nt).start()
            read_copy(t).wait()
            write_copy(t).start()
        for t in range(max(0, ntask - _DEPTH), ntask):
            write_copy(t).wait()

    return pl.pallas_call(
        body,
        in_specs=[pl.BlockSpec(memory_space=pl.ANY)],
        out_specs=pl.BlockSpec(memory_space=pl.ANY),
        out_shape=jax.ShapeDtypeStruct((s, b, c, out_len), dtype),
        scratch_shapes=[pltpu.VMEM((c, out_len), dtype)] * _DEPTH + [
            pltpu.SemaphoreType.DMA((_DEPTH,)),
            pltpu.SemaphoreType.DMA((_DEPTH,)),
        ],
    )


def kernel(wav):
    s, b, c, length = wav.shape
    out_len = length - _SHIFT
    return _make_copy(s, b, c, out_len, wav.dtype)(wav)
